# Initial kernel scaffold; baseline (speedup 1.0000x reference)
import jax
import jax.numpy as jnp
from jax.experimental import pallas as pl


def kernel(x, edge_index, W1, att_src1, att_dst1, b1, W2, att_src2, att_dst2, b2):
    raise NotImplementedError("write your pallas kernel here")


# speedup vs baseline: 12.2082x; 12.2082x over previous
import functools

import jax
import jax.numpy as jnp
from jax import lax
from jax.experimental import pallas as pl
from jax.experimental.pallas import tpu as pltpu
from jax.experimental.pallas import tpu_sc as plsc

_NEG = 0.2
_EPS = 1e-16
_L = 16
_K = 80


def _tc1_body(x_ref, w1_ref, acat_ref, h0_ref, h1_ref, h2_ref, h3_ref,
              a_ref):
  h = jnp.dot(x_ref[...], w1_ref[...], preferred_element_type=jnp.float32)
  h0_ref[...] = h[:, 0:32]
  h1_ref[...] = h[:, 32:64]
  h2_ref[...] = h[:, 64:96]
  h3_ref[...] = h[:, 96:128]
  a_ref[...] = jnp.dot(h, acat_ref[...], preferred_element_type=jnp.float32)


def _tc2_body(o00_ref, o01_ref, o02_ref, o03_ref, o10_ref, o11_ref, o12_ref,
              o13_ref, b1_ref, w2_ref, acat_ref, h2_ref, a2_ref):
  o = jnp.concatenate(
      [
          o00_ref[...] + o10_ref[...],
          o01_ref[...] + o11_ref[...],
          o02_ref[...] + o12_ref[...],
          o03_ref[...] + o13_ref[...],
      ],
      axis=1,
  ) + b1_ref[...]
  he = jnp.where(o > 0.0, o, jnp.exp(o) - 1.0)
  h2 = jnp.dot(he, w2_ref[...], preferred_element_type=jnp.float32)
  h2_ref[...] = h2
  a2_ref[...] = jnp.dot(h2, acat_ref[...], preferred_element_type=jnp.float32)


def _tc3_body(oa_ref, ob_ref, b2_ref, sel_ref, out_ref):
  o = oa_ref[...] + ob_ref[...]
  out_ref[...] = (
      jnp.dot(o, sel_ref[...], preferred_element_type=jnp.float32)
      + b2_ref[...]
  )


def _tc1(x, W1, acat):
  n = x.shape[0]
  d = x.shape[1]
  b = 1000
  return pl.pallas_call(
      _tc1_body,
      grid=(n // b,),
      in_specs=[
          pl.BlockSpec((b, d), lambda i: (i, 0)),
          pl.BlockSpec((d, 128), lambda i: (0, 0)),
          pl.BlockSpec((128, 16), lambda i: (0, 0)),
      ],
      out_specs=[pl.BlockSpec((b, 32), lambda i: (i, 0))] * 4
      + [pl.BlockSpec((b, 16), lambda i: (i, 0))],
      out_shape=[jax.ShapeDtypeStruct((n, 32), jnp.float32)] * 4
      + [jax.ShapeDtypeStruct((n, 16), jnp.float32)],
  )(x, W1, acat)


def _tc2(parts, b1, W2p, acat2):
  n = parts[0].shape[0]
  fp = W2p.shape[1]
  b = 1000
  quarter = pl.BlockSpec((b, 32), lambda i: (i, 0))
  return pl.pallas_call(
      _tc2_body,
      grid=(n // b,),
      in_specs=[quarter] * 8
      + [
          pl.BlockSpec((1, 128), lambda i: (0, 0)),
          pl.BlockSpec((128, fp), lambda i: (0, 0)),
          pl.BlockSpec((fp, 16), lambda i: (0, 0)),
      ],
      out_specs=[
          pl.BlockSpec((b, fp), lambda i: (i, 0)),
          pl.BlockSpec((b, 16), lambda i: (i, 0)),
      ],
      out_shape=[
          jax.ShapeDtypeStruct((n, fp), jnp.float32),
          jax.ShapeDtypeStruct((n, 16), jnp.float32),
      ],
  )(*parts, b1, W2p, acat2)


def _tc3(oa, ob, b2, sel):
  n = oa.shape[0]
  fp = oa.shape[1]
  c = sel.shape[1]
  b = 1000
  return pl.pallas_call(
      _tc3_body,
      grid=(n // b,),
      in_specs=[
          pl.BlockSpec((b, fp), lambda i: (i, 0)),
          pl.BlockSpec((b, fp), lambda i: (i, 0)),
          pl.BlockSpec((1, c), lambda i: (0, 0)),
          pl.BlockSpec((fp, c), lambda i: (0, 0)),
      ],
      out_specs=pl.BlockSpec((b, c), lambda i: (i, 0)),
      out_shape=jax.ShapeDtypeStruct((n, c), jnp.float32),
  )(oa, ob, b2, sel)


def _make_sc_gat(n, e, hh, fph, nph):
  info = plsc.get_sparse_core_info()
  nc, ns = info.num_cores, info.num_subcores
  nw = nc * ns
  hph = hh // nph
  ea = e // ns
  eb = e // nw
  ca = ea // _K
  cb = eb // _K
  nstr = 10
  rpt = n // nstr
  assert ea % _K == 0 and eb % _K == 0 and n % nstr == 0 and rpt % 8 == 0

  mesh = plsc.VectorSubcoreMesh(core_axis_name="c", subcore_axis_name="s")
  den_shape = (n, hh) if hh > 1 else (n,)
  w_shape = (_K, hh) if hh > 1 else (_K,)

  @functools.partial(
      pl.kernel,
      out_type=[
          jax.ShapeDtypeStruct((n, fph), jnp.float32)
          for _ in range(nc * nph)
      ],
      mesh=mesh,
      compiler_params=pltpu.CompilerParams(
          needs_layout_passes=False, use_tc_tiling_on_sc=False
      ),
      scratch_types=[
          pltpu.VMEM((_K,), jnp.int32),
          pltpu.VMEM((_K,), jnp.int32),
          pltpu.VMEM((_K, 16), jnp.float32),
          pltpu.VMEM((_K, 16), jnp.float32),
          pltpu.VMEM(w_shape, jnp.float32),
          pltpu.VMEM(w_shape, jnp.float32),
          pltpu.VMEM((_K, fph), jnp.float32),
          pltpu.VMEM(den_shape, jnp.float32),
          pltpu.VMEM_SHARED(den_shape, jnp.float32),
          pltpu.VMEM_SHARED((n, fph), jnp.float32),
      ],
  )
  def sc_layer(*refs):
    h_hbms = refs[:nph]
    a_hbm, src_hbm, dst_hbm, zbig_hbm, zden_hbm = refs[nph:nph + 5]
    out_hbms = refs[nph + 5:nph + 5 + nc * nph]
    (srcidx_v, dstidx_v, asrc_v, adst_v, w_v, alpha_v, hrows_v,
     den_v, den_sp, out_sp) = refs[nph + 5 + nc * nph:]

    cid = lax.axis_index("c")
    sid = lax.axis_index("s")
    wid = sid * nc + cid
    iota = lax.iota(jnp.int32, _L)
    row2 = iota >> 3
    col8 = iota & 7
    z16 = iota * 0
    o16 = z16 + 1

    def zero_stripes(sp_ref, z_ref):
      @pl.when(sid < nstr)
      def _():
        pltpu.sync_copy(
            z_ref.at[pl.ds(sid * rpt, rpt)], sp_ref.at[pl.ds(sid * rpt, rpt)]
        )

    def edge_w():
      ws = []
      if hh > 1:
        for p in range(_K // 2):
          rows = p * 2 + row2
          a_s = plsc.load_gather(asrc_v, [rows, col8])
          a_d = plsc.load_gather(adst_v, [rows, col8 + 8])
          ee = a_s + a_d
          ee = jnp.where(ee > 0.0, ee, _NEG * ee)
          ws.append((rows, col8, jnp.exp(ee)))
      else:
        for g in range(_K // _L):
          rows = g * _L + iota
          a_s = plsc.load_gather(asrc_v, [rows, z16])
          a_d = plsc.load_gather(adst_v, [rows, o16])
          ee = a_s + a_d
          ee = jnp.where(ee > 0.0, ee, _NEG * ee)
          ws.append((rows, None, jnp.exp(ee)))
      return ws

    zero_stripes(den_sp, zden_hbm)
    plsc.subcore_barrier()

    def pass_a(i, carry):
      base = sid * ea + i * _K
      pltpu.sync_copy(src_hbm.at[pl.ds(base, _K)], srcidx_v)
      pltpu.sync_copy(dst_hbm.at[pl.ds(base, _K)], dstidx_v)
      pltpu.sync_copy(a_hbm.at[srcidx_v], asrc_v)
      pltpu.sync_copy(a_hbm.at[dstidx_v], adst_v)
      for rows, cols, w in edge_w():
        if hh > 1:
          plsc.store_scatter(w_v, [rows, cols], w)
        else:
          plsc.store_scatter(w_v, [rows], w)
      pltpu.sync_copy(w_v, den_sp.at[dstidx_v], add=True)
      return carry

    lax.fori_loop(0, ca, pass_a, 0)
    plsc.subcore_barrier()

    pltpu.sync_copy(den_sp, den_v)

    for ph in range(nph):
      zero_stripes(out_sp, zbig_hbm)
      plsc.subcore_barrier()

      def pass_b(i, carry):
        base = wid * eb + i * _K
        pltpu.sync_copy(src_hbm.at[pl.ds(base, _K)], srcidx_v)
        pltpu.sync_copy(dst_hbm.at[pl.ds(base, _K)], dstidx_v)
        pltpu.sync_copy(a_hbm.at[srcidx_v], asrc_v)
        pltpu.sync_copy(a_hbm.at[dstidx_v], adst_v)
        pltpu.sync_copy(h_hbms[ph].at[srcidx_v], hrows_v)
        if hh > 1:
          for p in range(_K // 2):
            rows = p * 2 + row2
            a_s = plsc.load_gather(asrc_v, [rows, col8])
            a_d = plsc.load_gather(adst_v, [rows, col8 + 8])
            ee = a_s + a_d
            ee = jnp.where(ee > 0.0, ee, _NEG * ee)
            w = jnp.exp(ee)
            drows = plsc.load_gather(dstidx_v, [p * 2 + row2])
            den = plsc.load_gather(den_v, [drows, col8])
            alpha = w / (den + _EPS)
            plsc.store_scatter(alpha_v, [rows, col8], alpha)
        else:
          for g in range(_K // _L):
            rows = g * _L + iota
            a_s = plsc.load_gather(asrc_v, [rows, z16])
            a_d = plsc.load_gather(adst_v, [rows, o16])
            ee = a_s + a_d
            ee = jnp.where(ee > 0.0, ee, _NEG * ee)
            w = jnp.exp(ee)
            dsts = dstidx_v[pl.ds(g * _L, _L)]
            den = plsc.load_gather(den_v, [dsts])
            alpha = w / (den + _EPS)
            alpha_v[pl.ds(g * _L, _L)] = alpha

        def scale_edge(ei, c2):
          ei_v = z16 + ei
          if hh > 1:
            for head in range(hph):
              a = plsc.load_gather(alpha_v, [ei_v, z16 + (ph * hph + head)])
              sl = pl.ds(head * _L, _L)
              hrows_v[ei, sl] = hrows_v[ei, sl] * a
          else:
            a = plsc.load_gather(alpha_v, [ei_v])
            for j in range(fph // _L):
              sl = pl.ds(j * _L, _L)
              hrows_v[ei, sl] = hrows_v[ei, sl] * a
          return c2

        lax.fori_loop(0, _K, scale_edge, 0)
        pltpu.sync_copy(hrows_v, out_sp.at[dstidx_v], add=True)
        return carry

      lax.fori_loop(0, cb, pass_b, 0)
      plsc.subcore_barrier()

      for cc in range(nc):
        @pl.when((sid < nstr) & (cid == cc))
        def _():
          pltpu.sync_copy(
              out_sp.at[pl.ds(sid * rpt, rpt)],
              out_hbms[cc * nph + ph].at[pl.ds(sid * rpt, rpt)],
          )

      if ph + 1 < nph:
        plsc.subcore_barrier()

  return sc_layer


def kernel(x, edge_index, W1, att_src1, att_dst1, b1, W2, att_src2,
           att_dst2, b2):
  n, d = x.shape
  e = edge_index.shape[1]
  h, f = att_src1.shape
  c = W2.shape[1]
  fp2 = 48

  src = edge_index[0]
  dst = edge_index[1]

  eye_h = jnp.eye(h, dtype=jnp.float32)
  a1s = (eye_h[:, None, :] * att_src1[:, :, None]).reshape(h * f, h)
  a1d = (eye_h[:, None, :] * att_dst1[:, :, None]).reshape(h * f, h)
  acat1 = jnp.concatenate([a1s, a1d], axis=1)

  acat2 = jnp.zeros((fp2, 16), jnp.float32)
  acat2 = acat2.at[:c, 0].set(att_src2[0])
  acat2 = acat2.at[:c, 1].set(att_dst2[0])
  W2p = jnp.zeros((h * f, fp2), jnp.float32).at[:, :c].set(W2)
  sel = jnp.eye(fp2, c, dtype=jnp.float32)

  zbig1 = jnp.zeros((n, 32), jnp.float32)
  zden1 = jnp.zeros((n, h), jnp.float32)
  zbig2 = jnp.zeros((n, fp2), jnp.float32)
  zden2 = jnp.zeros((n,), jnp.float32)

  hq0, hq1, hq2, hq3, a1 = _tc1(x, W1, acat1)
  sc1 = _make_sc_gat(n, e, h, 32, 4)
  parts1 = sc1(hq0, hq1, hq2, hq3, a1, src, dst, zbig1, zden1)

  h2, a2 = _tc2(parts1, b1.reshape(1, -1), W2p, acat2)
  sc2 = _make_sc_gat(n, e, 1, fp2, 1)
  p20, p21 = sc2(h2, a2, src, dst, zbig2, zden2)

  return _tc3(p20, p21, b2.reshape(1, -1), sel)


# speedup vs baseline: 32.8730x; 2.6927x over previous
import functools

import jax
import jax.numpy as jnp
from jax import lax
from jax.experimental import pallas as pl
from jax.experimental.pallas import tpu as pltpu
from jax.experimental.pallas import tpu_sc as plsc

_NEG = 0.2
_EPS = 1e-16
_L = 16
_KC = 80
_KR = 5
_K = _KR * _KC


def _tc1_body(x_ref, w1_ref, acat_ref, h0_ref, h1_ref, a_ref):
  h = jnp.dot(x_ref[...], w1_ref[...], preferred_element_type=jnp.float32)
  h0_ref[...] = h[:, 0:64]
  h1_ref[...] = h[:, 64:128]
  a_ref[...] = jnp.dot(h, acat_ref[...], preferred_element_type=jnp.float32)


def _tc2_body(o00_ref, o01_ref, o10_ref, o11_ref, b1_ref, w2_ref, acat_ref,
              h2_ref, a2_ref):
  o = jnp.concatenate(
      [o00_ref[...] + o10_ref[...], o01_ref[...] + o11_ref[...]], axis=1
  ) + b1_ref[...]
  he = jnp.where(o > 0.0, o, jnp.exp(o) - 1.0)
  h2 = jnp.dot(he, w2_ref[...], preferred_element_type=jnp.float32)
  h2_ref[...] = h2
  a2_ref[...] = jnp.dot(h2, acat_ref[...], preferred_element_type=jnp.float32)


def _tc3_body(oa_ref, ob_ref, b2_ref, sel_ref, out_ref):
  o = oa_ref[...] + ob_ref[...]
  out_ref[...] = (
      jnp.dot(o, sel_ref[...], preferred_element_type=jnp.float32)
      + b2_ref[...]
  )


def _tc1(x, W1, acat):
  n = x.shape[0]
  d = x.shape[1]
  b = 1000
  return pl.pallas_call(
      _tc1_body,
      grid=(n // b,),
      in_specs=[
          pl.BlockSpec((b, d), lambda i: (i, 0)),
          pl.BlockSpec((d, 128), lambda i: (0, 0)),
          pl.BlockSpec((128, 16), lambda i: (0, 0)),
      ],
      out_specs=[pl.BlockSpec((b, 64), lambda i: (i, 0))] * 2
      + [pl.BlockSpec((b, 16), lambda i: (i, 0))],
      out_shape=[jax.ShapeDtypeStruct((n, 64), jnp.float32)] * 2
      + [jax.ShapeDtypeStruct((n, 16), jnp.float32)],
  )(x, W1, acat)


def _tc2(parts, b1, W2p, acat2):
  n = parts[0].shape[0]
  fp = W2p.shape[1]
  b = 1000
  quarter = pl.BlockSpec((b, 64), lambda i: (i, 0))
  return pl.pallas_call(
      _tc2_body,
      grid=(n // b,),
      in_specs=[quarter] * 4
      + [
          pl.BlockSpec((1, 128), lambda i: (0, 0)),
          pl.BlockSpec((128, fp), lambda i: (0, 0)),
          pl.BlockSpec((fp, 16), lambda i: (0, 0)),
      ],
      out_specs=[
          pl.BlockSpec((b, fp), lambda i: (i, 0)),
          pl.BlockSpec((b, 16), lambda i: (i, 0)),
      ],
      out_shape=[
          jax.ShapeDtypeStruct((n, fp), jnp.float32),
          jax.ShapeDtypeStruct((n, 16), jnp.float32),
      ],
  )(*parts, b1, W2p, acat2)


def _tc3(oa, ob, b2, sel):
  n = oa.shape[0]
  fp = oa.shape[1]
  c = sel.shape[1]
  b = 1000
  return pl.pallas_call(
      _tc3_body,
      grid=(n // b,),
      in_specs=[
          pl.BlockSpec((b, fp), lambda i: (i, 0)),
          pl.BlockSpec((b, fp), lambda i: (i, 0)),
          pl.BlockSpec((1, c), lambda i: (0, 0)),
          pl.BlockSpec((fp, c), lambda i: (0, 0)),
      ],
      out_specs=pl.BlockSpec((b, c), lambda i: (i, 0)),
      out_shape=jax.ShapeDtypeStruct((n, c), jnp.float32),
  )(oa, ob, b2, sel)


def _make_sc_gat(n, e, hh, fph, nph):
  info = plsc.get_sparse_core_info()
  nc, ns = info.num_cores, info.num_subcores
  nw = nc * ns
  hph = hh // nph
  ea = e // ns
  eb = e // nw
  ca = ea // _K
  cb = eb // _K
  nstr = 10
  rpt = n // nstr
  assert ea % _K == 0 and eb % _K == 0 and n % nstr == 0 and rpt % 8 == 0

  mesh = plsc.VectorSubcoreMesh(core_axis_name="c", subcore_axis_name="s")
  den_shape = (n, hh) if hh > 1 else (n,)
  w_shape = (_K, hh) if hh > 1 else (_K,)

  @functools.partial(
      pl.kernel,
      out_type=[
          jax.ShapeDtypeStruct((n, fph), jnp.float32)
          for _ in range(nc * nph)
      ]
      + [jax.ShapeDtypeStruct(den_shape, jnp.float32) for _ in range(nc)],
      mesh=mesh,
      compiler_params=pltpu.CompilerParams(
          needs_layout_passes=False, use_tc_tiling_on_sc=False
      ),
      scratch_types=[
          pltpu.VMEM((_KR, _KC), jnp.int32),
          pltpu.VMEM((_KR, _KC), jnp.int32),
          pltpu.VMEM((_K, 16), jnp.float32),
          pltpu.VMEM((_K, 16), jnp.float32),
          pltpu.VMEM(w_shape, jnp.float32),
          pltpu.VMEM(w_shape, jnp.float32),
          pltpu.VMEM((_K, fph), jnp.float32),
          pltpu.VMEM(w_shape, jnp.float32),
          pltpu.VMEM_SHARED(den_shape, jnp.float32),
          pltpu.VMEM_SHARED((n, fph), jnp.float32),
          pltpu.SemaphoreType.DMA,
      ],
  )
  def sc_layer(*refs):
    h_hbms = refs[:nph]
    a_hbm, src_hbm, dst_hbm, zbig_hbm, zden_hbm = refs[nph:nph + 5]
    out_hbms = refs[nph + 5:nph + 5 + nc * nph]
    den_hbms = refs[nph + 5 + nc * nph:nph + 5 + nc * nph + nc]
    (srcidx_v, dstidx_v, asrc_v, adst_v, w_v, alpha_v, hrows_v,
     den_rows_v, den_sp, out_sp, sem) = refs[nph + 5 + nc * (nph + 1):]

    def drain(cps):
      for cp in cps:
        cp.wait()

    def load_idx(row0):
      c1 = pltpu.async_copy(src_hbm.at[pl.ds(row0, _KR)], srcidx_v, sem)
      c2 = pltpu.async_copy(dst_hbm.at[pl.ds(row0, _KR)], dstidx_v, sem)
      drain([c1, c2])

    def gather_coeffs():
      cps = []
      for j in range(_KR):
        sl = pl.ds(j * _KC, _KC)
        cps.append(
            pltpu.async_copy(a_hbm.at[srcidx_v.at[j]], asrc_v.at[sl], sem)
        )
        cps.append(
            pltpu.async_copy(a_hbm.at[dstidx_v.at[j]], adst_v.at[sl], sem)
        )
      return cps

    def scatter_rows(src_ref, dst_ref):
      cps = []
      for j in range(_KR):
        cps.append(
            pltpu.async_copy(
                src_ref.at[pl.ds(j * _KC, _KC)],
                dst_ref.at[dstidx_v.at[j]],
                sem,
                add=True,
            )
        )
      drain(cps)

    cid = lax.axis_index("c")
    sid = lax.axis_index("s")
    wid = sid * nc + cid
    iota = lax.iota(jnp.int32, _L)
    row2 = iota >> 3
    col8 = iota & 7
    z16 = iota * 0
    o16 = z16 + 1

    def zero_stripes(sp_ref, z_ref):
      @pl.when(sid < nstr)
      def _():
        pltpu.sync_copy(
            z_ref.at[pl.ds(sid * rpt, rpt)], sp_ref.at[pl.ds(sid * rpt, rpt)]
        )

    def leaky_exp(a_s, a_d):
      ee = a_s + a_d
      return jnp.exp(jnp.where(ee > 0.0, ee, _NEG * ee))

    zero_stripes(den_sp, zden_hbm)
    plsc.subcore_barrier()

    def w_chunk():
      if hh > 1:
        def wpair(p, c2):
          rows = p * 2 + row2
          a_s = plsc.load_gather(asrc_v, [rows, col8])
          a_d = plsc.load_gather(adst_v, [rows, col8 + 8])
          plsc.store_scatter(w_v, [rows, col8], leaky_exp(a_s, a_d))
          return c2

        lax.fori_loop(0, _K // 2, wpair, 0)
      else:
        def wgrp(g, c2):
          rows = g * _L + iota
          a_s = plsc.load_gather(asrc_v, [rows, z16])
          a_d = plsc.load_gather(adst_v, [rows, o16])
          plsc.store_scatter(w_v, [rows], leaky_exp(a_s, a_d))
          return c2

        lax.fori_loop(0, _K // _L, wgrp, 0)

    def pass_a(i, carry):
      load_idx(sid * (ea // _KC) + i * _KR)
      drain(gather_coeffs())
      w_chunk()
      scatter_rows(w_v, den_sp)
      return carry

    lax.fori_loop(0, ca, pass_a, 0)
    plsc.subcore_barrier()

    for cc in range(nc):
      @pl.when((sid < nstr) & (cid == cc))
      def _(cc=cc):
        pltpu.sync_copy(
            den_sp.at[pl.ds(sid * rpt, rpt)],
            den_hbms[cc].at[pl.ds(sid * rpt, rpt)],
        )
    plsc.subcore_barrier()

    for ph in range(nph):
      zero_stripes(out_sp, zbig_hbm)
      plsc.subcore_barrier()

      def pass_b(i, carry):
        load_idx(wid * (eb // _KC) + i * _KR)
        cps = gather_coeffs()
        for j in range(_KR):
          cps.append(
              pltpu.async_copy(
                  h_hbms[ph].at[srcidx_v.at[j]],
                  hrows_v.at[pl.ds(j * _KC, _KC)],
                  sem,
              )
          )
        for cc in range(nc):
          @pl.when(cid == cc)
          def _(cc=cc):
            dcps = []
            for j in range(_KR):
              dcps.append(
                  pltpu.async_copy(
                      den_hbms[cc].at[dstidx_v.at[j]],
                      den_rows_v.at[pl.ds(j * _KC, _KC)],
                      sem,
                  )
              )
            drain(dcps)
        drain(cps)
        if hh > 1:
          def apair(p, c2):
            rows = p * 2 + row2
            a_s = plsc.load_gather(asrc_v, [rows, col8])
            a_d = plsc.load_gather(adst_v, [rows, col8 + 8])
            w = leaky_exp(a_s, a_d)
            den = plsc.load_gather(den_rows_v, [rows, col8])
            plsc.store_scatter(alpha_v, [rows, col8], w / (den + _EPS))
            return c2

          lax.fori_loop(0, _K // 2, apair, 0)
        else:
          def agrp(g, c2):
            rows = g * _L + iota
            a_s = plsc.load_gather(asrc_v, [rows, z16])
            a_d = plsc.load_gather(adst_v, [rows, o16])
            w = leaky_exp(a_s, a_d)
            den = plsc.load_gather(den_rows_v, [rows])
            plsc.store_scatter(alpha_v, [rows], w / (den + _EPS))
            return c2

          lax.fori_loop(0, _K // _L, agrp, 0)

        def scale_edge(ei, c2):
          ei_v = z16 + ei
          if hh > 1:
            for head in range(hph):
              a = plsc.load_gather(alpha_v, [ei_v, z16 + (ph * hph + head)])
              sl = pl.ds(head * _L, _L)
              hrows_v[ei, sl] = hrows_v[ei, sl] * a
          else:
            a = plsc.load_gather(alpha_v, [ei_v])
            for j in range(fph // _L):
              sl = pl.ds(j * _L, _L)
              hrows_v[ei, sl] = hrows_v[ei, sl] * a
          return c2

        lax.fori_loop(0, _K, scale_edge, 0)
        scatter_rows(hrows_v, out_sp)
        return carry

      lax.fori_loop(0, cb, pass_b, 0)
      plsc.subcore_barrier()

      for cc in range(nc):
        @pl.when((sid < nstr) & (cid == cc))
        def _():
          pltpu.sync_copy(
              out_sp.at[pl.ds(sid * rpt, rpt)],
              out_hbms[cc * nph + ph].at[pl.ds(sid * rpt, rpt)],
          )

      if ph + 1 < nph:
        plsc.subcore_barrier()

  return sc_layer


def kernel(x, edge_index, W1, att_src1, att_dst1, b1, W2, att_src2,
           att_dst2, b2):
  n, d = x.shape
  e = edge_index.shape[1]
  h, f = att_src1.shape
  c = W2.shape[1]
  fp2 = 48

  src = edge_index[0].reshape(e // _KC, _KC)
  dst = edge_index[1].reshape(e // _KC, _KC)

  eye_h = jnp.eye(h, dtype=jnp.float32)
  a1s = (eye_h[:, None, :] * att_src1[:, :, None]).reshape(h * f, h)
  a1d = (eye_h[:, None, :] * att_dst1[:, :, None]).reshape(h * f, h)
  acat1 = jnp.concatenate([a1s, a1d], axis=1)

  acat2 = jnp.zeros((fp2, 16), jnp.float32)
  acat2 = acat2.at[:c, 0].set(att_src2[0])
  acat2 = acat2.at[:c, 1].set(att_dst2[0])
  W2p = jnp.zeros((h * f, fp2), jnp.float32).at[:, :c].set(W2)
  sel = jnp.eye(fp2, c, dtype=jnp.float32)

  zbig1 = jnp.zeros((n, 64), jnp.float32)
  zden1 = jnp.zeros((n, h), jnp.float32)
  zbig2 = jnp.zeros((n, fp2), jnp.float32)
  zden2 = jnp.zeros((n,), jnp.float32)

  hlo, hhi, a1 = _tc1(x, W1, acat1)
  sc1 = _make_sc_gat(n, e, h, 64, 2)
  *parts1, _, _ = sc1(hlo, hhi, a1, src, dst, zbig1, zden1)

  h2, a2 = _tc2(parts1, b1.reshape(1, -1), W2p, acat2)
  sc2 = _make_sc_gat(n, e, 1, fp2, 1)
  p20, p21, _, _ = sc2(h2, a2, src, dst, zbig2, zden2)

  return _tc3(p20, p21, b2.reshape(1, -1), sel)


# speedup vs baseline: 55.9298x; 1.7014x over previous
import functools

import jax
import jax.numpy as jnp
from jax import lax
from jax.experimental import pallas as pl
from jax.experimental.pallas import tpu as pltpu
from jax.experimental.pallas import tpu_sc as plsc

_NEG = 0.2
_EPS = 1e-16
_L = 16
_KC = 80
_KR = 5
_K = _KR * _KC


def _tc1_body(x_ref, w1_ref, acat_ref, h0_ref, h1_ref, a_ref):
  h = jnp.dot(x_ref[...], w1_ref[...], preferred_element_type=jnp.float32)
  h0_ref[...] = h[:, 0:64]
  h1_ref[...] = h[:, 64:128]
  a_ref[...] = jnp.dot(h, acat_ref[...], preferred_element_type=jnp.float32)


def _tc2_body(o00_ref, o01_ref, o10_ref, o11_ref, b1_ref, w2_ref, acat_ref,
              h2_ref, a2_ref):
  o = jnp.concatenate(
      [o00_ref[...] + o10_ref[...], o01_ref[...] + o11_ref[...]], axis=1
  ) + b1_ref[...]
  he = jnp.where(o > 0.0, o, jnp.exp(o) - 1.0)
  h2 = jnp.dot(he, w2_ref[...], preferred_element_type=jnp.float32)
  h2_ref[...] = h2
  a2_ref[...] = jnp.dot(h2, acat_ref[...], preferred_element_type=jnp.float32)


def _tc3_body(oa_ref, ob_ref, b2_ref, sel_ref, out_ref):
  o = oa_ref[...] + ob_ref[...]
  out_ref[...] = (
      jnp.dot(o, sel_ref[...], preferred_element_type=jnp.float32)
      + b2_ref[...]
  )


def _tc1(x, W1, acat):
  n = x.shape[0]
  d = x.shape[1]
  b = 1000
  return pl.pallas_call(
      _tc1_body,
      grid=(n // b,),
      in_specs=[
          pl.BlockSpec((b, d), lambda i: (i, 0)),
          pl.BlockSpec((d, 128), lambda i: (0, 0)),
          pl.BlockSpec((128, 16), lambda i: (0, 0)),
      ],
      out_specs=[pl.BlockSpec((b, 64), lambda i: (i, 0))] * 2
      + [pl.BlockSpec((b, 16), lambda i: (i, 0))],
      out_shape=[jax.ShapeDtypeStruct((n, 64), jnp.float32)] * 2
      + [jax.ShapeDtypeStruct((n, 16), jnp.float32)],
  )(x, W1, acat)


def _tc2(parts, b1, W2p, acat2):
  n = parts[0].shape[0]
  fp = W2p.shape[1]
  b = 1000
  quarter = pl.BlockSpec((b, 64), lambda i: (i, 0))
  return pl.pallas_call(
      _tc2_body,
      grid=(n // b,),
      in_specs=[quarter] * 4
      + [
          pl.BlockSpec((1, 128), lambda i: (0, 0)),
          pl.BlockSpec((128, fp), lambda i: (0, 0)),
          pl.BlockSpec((fp, 16), lambda i: (0, 0)),
      ],
      out_specs=[
          pl.BlockSpec((b, fp), lambda i: (i, 0)),
          pl.BlockSpec((b, 16), lambda i: (i, 0)),
      ],
      out_shape=[
          jax.ShapeDtypeStruct((n, fp), jnp.float32),
          jax.ShapeDtypeStruct((n, 16), jnp.float32),
      ],
  )(*parts, b1, W2p, acat2)


def _tc3(oa, ob, b2, sel):
  n = oa.shape[0]
  fp = oa.shape[1]
  c = sel.shape[1]
  b = 1000
  return pl.pallas_call(
      _tc3_body,
      grid=(n // b,),
      in_specs=[
          pl.BlockSpec((b, fp), lambda i: (i, 0)),
          pl.BlockSpec((b, fp), lambda i: (i, 0)),
          pl.BlockSpec((1, c), lambda i: (0, 0)),
          pl.BlockSpec((fp, c), lambda i: (0, 0)),
      ],
      out_specs=pl.BlockSpec((b, c), lambda i: (i, 0)),
      out_shape=jax.ShapeDtypeStruct((n, c), jnp.float32),
  )(oa, ob, b2, sel)


def _make_sc_gat(n, e, hh, fph, nph):
  info = plsc.get_sparse_core_info()
  nc, ns = info.num_cores, info.num_subcores
  nw = nc * ns
  hph = hh // nph
  ea = e // ns
  eb = e // nw
  ca = ea // _K
  cb = eb // _K
  nstr = 10
  rpt = n // nstr
  assert ea % _K == 0 and eb % _K == 0 and n % nstr == 0 and rpt % 8 == 0

  mesh = plsc.VectorSubcoreMesh(core_axis_name="c", subcore_axis_name="s")
  den_shape = (n, hh) if hh > 1 else (n,)
  w_shape = (_K, hh) if hh > 1 else (_K,)

  @functools.partial(
      pl.kernel,
      out_type=[
          jax.ShapeDtypeStruct((n, fph), jnp.float32)
          for _ in range(nc * nph)
      ]
      + [jax.ShapeDtypeStruct(den_shape, jnp.float32) for _ in range(nc)],
      mesh=mesh,
      compiler_params=pltpu.CompilerParams(
          needs_layout_passes=False, use_tc_tiling_on_sc=False
      ),
      scratch_types=[
          pltpu.VMEM((_KR, _KC), jnp.int32),
          pltpu.VMEM((_KR, _KC), jnp.int32),
          pltpu.VMEM((_K, 16), jnp.float32),
          pltpu.VMEM((_K, 16), jnp.float32),
          pltpu.VMEM(w_shape, jnp.float32),
          pltpu.VMEM(w_shape, jnp.float32),
          pltpu.VMEM((_K, fph), jnp.float32),
          pltpu.VMEM(w_shape, jnp.float32),
          pltpu.VMEM_SHARED(den_shape, jnp.float32),
          pltpu.VMEM_SHARED((n, fph), jnp.float32),
          pltpu.SemaphoreType.DMA,
      ],
  )
  def sc_layer(*refs):
    h_hbms = refs[:nph]
    a_hbm, src_hbm, dst_hbm, zbig_hbm, zden_hbm = refs[nph:nph + 5]
    out_hbms = refs[nph + 5:nph + 5 + nc * nph]
    den_hbms = refs[nph + 5 + nc * nph:nph + 5 + nc * nph + nc]
    (srcidx_v, dstidx_v, asrc_v, adst_v, w_v, alpha_v, hrows_v,
     den_rows_v, den_sp, out_sp, sem) = refs[nph + 5 + nc * (nph + 1):]

    def drain(cps):
      for cp in cps:
        cp.wait()

    def load_idx(row0):
      c1 = pltpu.async_copy(src_hbm.at[pl.ds(row0, _KR)], srcidx_v, sem)
      c2 = pltpu.async_copy(dst_hbm.at[pl.ds(row0, _KR)], dstidx_v, sem)
      drain([c1, c2])

    def gather_coeffs():
      cps = []
      for j in range(_KR):
        sl = pl.ds(j * _KC, _KC)
        cps.append(
            pltpu.async_copy(a_hbm.at[srcidx_v.at[j]], asrc_v.at[sl], sem)
        )
        cps.append(
            pltpu.async_copy(a_hbm.at[dstidx_v.at[j]], adst_v.at[sl], sem)
        )
      return cps

    def scatter_rows(src_ref, dst_ref):
      cps = []
      for j in range(_KR):
        cps.append(
            pltpu.async_copy(
                src_ref.at[pl.ds(j * _KC, _KC)],
                dst_ref.at[dstidx_v.at[j]],
                sem,
                add=True,
            )
        )
      drain(cps)

    cid = lax.axis_index("c")
    sid = lax.axis_index("s")
    wid = sid * nc + cid
    iota = lax.iota(jnp.int32, _L)
    row2 = iota >> 3
    col8 = iota & 7
    z16 = iota * 0
    o16 = z16 + 1

    def zero_stripes(sp_ref, z_ref):
      @pl.when(sid < nstr)
      def _():
        pltpu.sync_copy(
            z_ref.at[pl.ds(sid * rpt, rpt)], sp_ref.at[pl.ds(sid * rpt, rpt)]
        )

    def leaky_exp(a_s, a_d):
      ee = a_s + a_d
      return jnp.exp(jnp.where(ee > 0.0, ee, _NEG * ee))

    zero_stripes(den_sp, zden_hbm)
    plsc.subcore_barrier()

    def w_chunk():
      if hh > 1:
        @plsc.parallel_loop(0, _K // 2, unroll=4)
        def _(p):
          rows = p * 2 + row2
          a_s = plsc.load_gather(asrc_v, [rows, col8])
          a_d = plsc.load_gather(adst_v, [rows, col8 + 8])
          plsc.store_scatter(w_v, [rows, col8], leaky_exp(a_s, a_d))
      else:
        @plsc.parallel_loop(0, _K // _L, unroll=4)
        def _(g):
          rows = g * _L + iota
          a_s = plsc.load_gather(asrc_v, [rows, z16])
          a_d = plsc.load_gather(adst_v, [rows, o16])
          plsc.store_scatter(w_v, [rows], leaky_exp(a_s, a_d))

    def pass_a(i, carry):
      load_idx(sid * (ea // _KC) + i * _KR)
      drain(gather_coeffs())
      w_chunk()
      scatter_rows(w_v, den_sp)
      return carry

    lax.fori_loop(0, ca, pass_a, 0)
    plsc.subcore_barrier()

    for cc in range(nc):
      @pl.when((sid < nstr) & (cid == cc))
      def _(cc=cc):
        pltpu.sync_copy(
            den_sp.at[pl.ds(sid * rpt, rpt)],
            den_hbms[cc].at[pl.ds(sid * rpt, rpt)],
        )
    plsc.subcore_barrier()

    for ph in range(nph):
      zero_stripes(out_sp, zbig_hbm)
      plsc.subcore_barrier()

      def pass_b(i, carry):
        load_idx(wid * (eb // _KC) + i * _KR)
        cps = gather_coeffs()
        for j in range(_KR):
          cps.append(
              pltpu.async_copy(
                  h_hbms[ph].at[srcidx_v.at[j]],
                  hrows_v.at[pl.ds(j * _KC, _KC)],
                  sem,
              )
          )
        for cc in range(nc):
          @pl.when(cid == cc)
          def _(cc=cc):
            dcps = []
            for j in range(_KR):
              dcps.append(
                  pltpu.async_copy(
                      den_hbms[cc].at[dstidx_v.at[j]],
                      den_rows_v.at[pl.ds(j * _KC, _KC)],
                      sem,
                  )
              )
            drain(dcps)
        drain(cps)
        if hh > 1:
          @plsc.parallel_loop(0, _K // 2, unroll=4)
          def _(p):
            rows = p * 2 + row2
            a_s = plsc.load_gather(asrc_v, [rows, col8])
            a_d = plsc.load_gather(adst_v, [rows, col8 + 8])
            w = leaky_exp(a_s, a_d)
            den = plsc.load_gather(den_rows_v, [rows, col8])
            plsc.store_scatter(alpha_v, [rows, col8], w / (den + _EPS))
        else:
          @plsc.parallel_loop(0, _K // _L, unroll=4)
          def _(g):
            rows = g * _L + iota
            a_s = plsc.load_gather(asrc_v, [rows, z16])
            a_d = plsc.load_gather(adst_v, [rows, o16])
            w = leaky_exp(a_s, a_d)
            den = plsc.load_gather(den_rows_v, [rows])
            plsc.store_scatter(alpha_v, [rows], w / (den + _EPS))

        @plsc.parallel_loop(0, _K, unroll=4)
        def _(ei):
          ei_v = z16 + ei
          if hh > 1:
            for head in range(hph):
              a = plsc.load_gather(alpha_v, [ei_v, z16 + (ph * hph + head)])
              sl = pl.ds(head * _L, _L)
              hrows_v[ei, sl] = hrows_v[ei, sl] * a
          else:
            a = plsc.load_gather(alpha_v, [ei_v])
            for j in range(fph // _L):
              sl = pl.ds(j * _L, _L)
              hrows_v[ei, sl] = hrows_v[ei, sl] * a
        scatter_rows(hrows_v, out_sp)
        return carry

      lax.fori_loop(0, cb, pass_b, 0)
      plsc.subcore_barrier()

      for cc in range(nc):
        @pl.when((sid < nstr) & (cid == cc))
        def _():
          pltpu.sync_copy(
              out_sp.at[pl.ds(sid * rpt, rpt)],
              out_hbms[cc * nph + ph].at[pl.ds(sid * rpt, rpt)],
          )

      if ph + 1 < nph:
        plsc.subcore_barrier()

  return sc_layer


def kernel(x, edge_index, W1, att_src1, att_dst1, b1, W2, att_src2,
           att_dst2, b2):
  n, d = x.shape
  e = edge_index.shape[1]
  h, f = att_src1.shape
  c = W2.shape[1]
  fp2 = 48

  src = edge_index[0].reshape(e // _KC, _KC)
  dst = edge_index[1].reshape(e // _KC, _KC)

  eye_h = jnp.eye(h, dtype=jnp.float32)
  a1s = (eye_h[:, None, :] * att_src1[:, :, None]).reshape(h * f, h)
  a1d = (eye_h[:, None, :] * att_dst1[:, :, None]).reshape(h * f, h)
  acat1 = jnp.concatenate([a1s, a1d], axis=1)

  acat2 = jnp.zeros((fp2, 16), jnp.float32)
  acat2 = acat2.at[:c, 0].set(att_src2[0])
  acat2 = acat2.at[:c, 1].set(att_dst2[0])
  W2p = jnp.zeros((h * f, fp2), jnp.float32).at[:, :c].set(W2)
  sel = jnp.eye(fp2, c, dtype=jnp.float32)

  zbig1 = jnp.zeros((n, 64), jnp.float32)
  zden1 = jnp.zeros((n, h), jnp.float32)
  zbig2 = jnp.zeros((n, fp2), jnp.float32)
  zden2 = jnp.zeros((n,), jnp.float32)

  hlo, hhi, a1 = _tc1(x, W1, acat1)
  sc1 = _make_sc_gat(n, e, h, 64, 2)
  *parts1, _, _ = sc1(hlo, hhi, a1, src, dst, zbig1, zden1)

  h2, a2 = _tc2(parts1, b1.reshape(1, -1), W2p, acat2)
  sc2 = _make_sc_gat(n, e, 1, fp2, 1)
  p20, p21, _, _ = sc2(h2, a2, src, dst, zbig2, zden2)

  return _tc3(p20, p21, b2.reshape(1, -1), sel)


# speedup vs baseline: 60.4477x; 1.0808x over previous
import functools

import jax
import jax.numpy as jnp
from jax import lax
from jax.experimental import pallas as pl
from jax.experimental.pallas import tpu as pltpu
from jax.experimental.pallas import tpu_sc as plsc

_NEG = 0.2
_EPS = 1e-16
_L = 16
_KC = 80
_KR = 5
_K = _KR * _KC


def _tc1_body(x_ref, w1_ref, acat_ref, h0_ref, h1_ref, a_ref):
  h = jnp.dot(x_ref[...], w1_ref[...], preferred_element_type=jnp.float32)
  h0_ref[...] = h[:, 0:64]
  h1_ref[...] = h[:, 64:128]
  a_ref[...] = jnp.dot(h, acat_ref[...], preferred_element_type=jnp.float32)


def _tc2_body(o00_ref, o01_ref, o10_ref, o11_ref, b1_ref, w2_ref, acat_ref,
              h2_ref, a2_ref):
  o = jnp.concatenate(
      [o00_ref[...] + o10_ref[...], o01_ref[...] + o11_ref[...]], axis=1
  ) + b1_ref[...]
  he = jnp.where(o > 0.0, o, jnp.exp(o) - 1.0)
  h2 = jnp.dot(he, w2_ref[...], preferred_element_type=jnp.float32)
  h2_ref[...] = h2
  a2_ref[...] = jnp.dot(h2, acat_ref[...], preferred_element_type=jnp.float32)


def _tc3_body(oa_ref, ob_ref, b2_ref, sel_ref, out_ref):
  o = oa_ref[...] + ob_ref[...]
  out_ref[...] = (
      jnp.dot(o, sel_ref[...], preferred_element_type=jnp.float32)
      + b2_ref[...]
  )


def _tc1(x, W1, acat):
  n = x.shape[0]
  d = x.shape[1]
  b = 1000
  return pl.pallas_call(
      _tc1_body,
      grid=(n // b,),
      in_specs=[
          pl.BlockSpec((b, d), lambda i: (i, 0)),
          pl.BlockSpec((d, 128), lambda i: (0, 0)),
          pl.BlockSpec((128, 16), lambda i: (0, 0)),
      ],
      out_specs=[pl.BlockSpec((b, 64), lambda i: (i, 0))] * 2
      + [pl.BlockSpec((b, 16), lambda i: (i, 0))],
      out_shape=[jax.ShapeDtypeStruct((n, 64), jnp.float32)] * 2
      + [jax.ShapeDtypeStruct((n, 16), jnp.float32)],
  )(x, W1, acat)


def _tc2(parts, b1, W2p, acat2):
  n = parts[0].shape[0]
  fp = W2p.shape[1]
  b = 1000
  quarter = pl.BlockSpec((b, 64), lambda i: (i, 0))
  return pl.pallas_call(
      _tc2_body,
      grid=(n // b,),
      in_specs=[quarter] * 4
      + [
          pl.BlockSpec((1, 128), lambda i: (0, 0)),
          pl.BlockSpec((128, fp), lambda i: (0, 0)),
          pl.BlockSpec((fp, 16), lambda i: (0, 0)),
      ],
      out_specs=[
          pl.BlockSpec((b, fp), lambda i: (i, 0)),
          pl.BlockSpec((b, 16), lambda i: (i, 0)),
      ],
      out_shape=[
          jax.ShapeDtypeStruct((n, fp), jnp.float32),
          jax.ShapeDtypeStruct((n, 16), jnp.float32),
      ],
  )(*parts, b1, W2p, acat2)


def _tc3(oa, ob, b2, sel):
  n = oa.shape[0]
  fp = oa.shape[1]
  c = sel.shape[1]
  b = 1000
  return pl.pallas_call(
      _tc3_body,
      grid=(n // b,),
      in_specs=[
          pl.BlockSpec((b, fp), lambda i: (i, 0)),
          pl.BlockSpec((b, fp), lambda i: (i, 0)),
          pl.BlockSpec((1, c), lambda i: (0, 0)),
          pl.BlockSpec((fp, c), lambda i: (0, 0)),
      ],
      out_specs=pl.BlockSpec((b, c), lambda i: (i, 0)),
      out_shape=jax.ShapeDtypeStruct((n, c), jnp.float32),
  )(oa, ob, b2, sel)


def _make_sc_gat(n, e, hh, fph, nph):
  info = plsc.get_sparse_core_info()
  nc, ns = info.num_cores, info.num_subcores
  nw = nc * ns
  hph = hh // nph
  ea = e // ns
  eb = e // nw
  ca = ea // _K
  cb = eb // _K
  nstr = 10
  rpt = n // nstr
  assert ea % _K == 0 and eb % _K == 0 and n % nstr == 0 and rpt % 8 == 0

  mesh = plsc.VectorSubcoreMesh(core_axis_name="c", subcore_axis_name="s")
  den_shape = (n, hh) if hh > 1 else (n,)
  w_shape = (_K, hh) if hh > 1 else (_K,)

  @functools.partial(
      pl.kernel,
      out_type=[
          jax.ShapeDtypeStruct((n, fph), jnp.float32)
          for _ in range(nc * nph)
      ]
      + [jax.ShapeDtypeStruct(den_shape, jnp.float32) for _ in range(nc)]
      + [
          jax.ShapeDtypeStruct((e, hh) if hh > 1 else (e,), jnp.float32)
          for _ in range(nc)
      ],
      mesh=mesh,
      compiler_params=pltpu.CompilerParams(
          needs_layout_passes=False, use_tc_tiling_on_sc=False
      ),
      scratch_types=[
          pltpu.VMEM((_KR, _KC), jnp.int32),
          pltpu.VMEM((_KR, _KC), jnp.int32),
          pltpu.VMEM((_K, 16), jnp.float32),
          pltpu.VMEM((_K, 16), jnp.float32),
          pltpu.VMEM(w_shape, jnp.float32),
          pltpu.VMEM(w_shape, jnp.float32),
          pltpu.VMEM((_K, fph), jnp.float32),
          pltpu.VMEM(w_shape, jnp.float32),
          pltpu.VMEM_SHARED(den_shape, jnp.float32),
          pltpu.VMEM_SHARED((n, fph), jnp.float32),
          pltpu.SemaphoreType.DMA,
      ],
  )
  def sc_layer(*refs):
    h_hbms = refs[:nph]
    a_hbm, src_hbm, dst_hbm, zbig_hbm, zden_hbm = refs[nph:nph + 5]
    out_hbms = refs[nph + 5:nph + 5 + nc * nph]
    den_hbms = refs[nph + 5 + nc * nph:nph + 5 + nc * nph + nc]
    w_hbms = refs[nph + 5 + nc * (nph + 1):nph + 5 + nc * (nph + 2)]
    (srcidx_v, dstidx_v, asrc_v, adst_v, w_v, alpha_v, hrows_v,
     den_rows_v, den_sp, out_sp, sem) = refs[nph + 5 + nc * (nph + 2):]

    def drain(cps):
      for cp in cps:
        cp.wait()

    def load_idx(row0):
      c1 = pltpu.async_copy(src_hbm.at[pl.ds(row0, _KR)], srcidx_v, sem)
      c2 = pltpu.async_copy(dst_hbm.at[pl.ds(row0, _KR)], dstidx_v, sem)
      drain([c1, c2])

    def gather_coeffs():
      cps = []
      for j in range(_KR):
        sl = pl.ds(j * _KC, _KC)
        cps.append(
            pltpu.async_copy(a_hbm.at[srcidx_v.at[j]], asrc_v.at[sl], sem)
        )
        cps.append(
            pltpu.async_copy(a_hbm.at[dstidx_v.at[j]], adst_v.at[sl], sem)
        )
      return cps

    def scatter_rows(src_ref, dst_ref):
      cps = []
      for j in range(_KR):
        cps.append(
            pltpu.async_copy(
                src_ref.at[pl.ds(j * _KC, _KC)],
                dst_ref.at[dstidx_v.at[j]],
                sem,
                add=True,
            )
        )
      drain(cps)

    cid = lax.axis_index("c")
    sid = lax.axis_index("s")
    wid = sid * nc + cid
    iota = lax.iota(jnp.int32, _L)
    row2 = iota >> 3
    col8 = iota & 7
    z16 = iota * 0
    o16 = z16 + 1

    def zero_stripes(sp_ref, z_ref):
      @pl.when(sid < nstr)
      def _():
        pltpu.sync_copy(
            z_ref.at[pl.ds(sid * rpt, rpt)], sp_ref.at[pl.ds(sid * rpt, rpt)]
        )

    def leaky_exp(a_s, a_d):
      ee = a_s + a_d
      return jnp.exp(jnp.where(ee > 0.0, ee, _NEG * ee))

    zero_stripes(den_sp, zden_hbm)
    plsc.subcore_barrier()

    def w_chunk():
      if hh > 1:
        @plsc.parallel_loop(0, _K // 2, unroll=4)
        def _(p):
          rows = p * 2 + row2
          a_s = plsc.load_gather(asrc_v, [rows, col8])
          a_d = plsc.load_gather(adst_v, [rows, col8 + 8])
          plsc.store_scatter(w_v, [rows, col8], leaky_exp(a_s, a_d))
      else:
        @plsc.parallel_loop(0, _K // _L, unroll=4)
        def _(g):
          rows = g * _L + iota
          a_s = plsc.load_gather(asrc_v, [rows, z16])
          a_d = plsc.load_gather(adst_v, [rows, o16])
          plsc.store_scatter(w_v, [rows], leaky_exp(a_s, a_d))

    def pass_a(i, carry):
      load_idx(sid * (ea // _KC) + i * _KR)
      drain(gather_coeffs())
      w_chunk()
      for cc in range(nc):
        @pl.when(cid == cc)
        def _(cc=cc):
          ebase = sid * ea + i * _K
          pltpu.async_copy(
              w_v, w_hbms[cc].at[pl.ds(ebase, _K)], sem
          ).wait()
      scatter_rows(w_v, den_sp)
      return carry

    lax.fori_loop(0, ca, pass_a, 0)
    plsc.subcore_barrier()

    for cc in range(nc):
      @pl.when((sid < nstr) & (cid == cc))
      def _(cc=cc):
        pltpu.sync_copy(
            den_sp.at[pl.ds(sid * rpt, rpt)],
            den_hbms[cc].at[pl.ds(sid * rpt, rpt)],
        )
    plsc.subcore_barrier()

    for ph in range(nph):
      zero_stripes(out_sp, zbig_hbm)
      plsc.subcore_barrier()

      def pass_b(i, carry):
        ebase = wid * eb + i * _K
        load_idx(wid * (eb // _KC) + i * _KR)
        cps = []
        for j in range(_KR):
          cps.append(
              pltpu.async_copy(
                  h_hbms[ph].at[srcidx_v.at[j]],
                  hrows_v.at[pl.ds(j * _KC, _KC)],
                  sem,
              )
          )
        for cc in range(nc):
          @pl.when(cid == cc)
          def _(cc=cc):
            dcps = [
                pltpu.async_copy(
                    w_hbms[cc].at[pl.ds(ebase, _K)], w_v, sem
                )
            ]
            for j in range(_KR):
              dcps.append(
                  pltpu.async_copy(
                      den_hbms[cc].at[dstidx_v.at[j]],
                      den_rows_v.at[pl.ds(j * _KC, _KC)],
                      sem,
                  )
              )
            drain(dcps)
        drain(cps)
        if hh > 1:
          @plsc.parallel_loop(0, _K // 2, unroll=4)
          def _(p):
            rows = p * 2 + row2
            w = plsc.load_gather(w_v, [rows, col8])
            den = plsc.load_gather(den_rows_v, [rows, col8])
            plsc.store_scatter(alpha_v, [rows, col8], w / (den + _EPS))
        else:
          @plsc.parallel_loop(0, _K // _L, unroll=4)
          def _(g):
            rows = g * _L + iota
            w = plsc.load_gather(w_v, [rows])
            den = plsc.load_gather(den_rows_v, [rows])
            plsc.store_scatter(alpha_v, [rows], w / (den + _EPS))

        @plsc.parallel_loop(0, _K, unroll=8)
        def _(ei):
          ei_v = z16 + ei
          if hh > 1:
            for head in range(hph):
              a = plsc.load_gather(alpha_v, [ei_v, z16 + (ph * hph + head)])
              sl = pl.ds(head * _L, _L)
              hrows_v[ei, sl] = hrows_v[ei, sl] * a
          else:
            a = plsc.load_gather(alpha_v, [ei_v])
            for j in range(fph // _L):
              sl = pl.ds(j * _L, _L)
              hrows_v[ei, sl] = hrows_v[ei, sl] * a
        scatter_rows(hrows_v, out_sp)
        return carry

      lax.fori_loop(0, cb, pass_b, 0)
      plsc.subcore_barrier()

      for cc in range(nc):
        @pl.when((sid < nstr) & (cid == cc))
        def _():
          pltpu.sync_copy(
              out_sp.at[pl.ds(sid * rpt, rpt)],
              out_hbms[cc * nph + ph].at[pl.ds(sid * rpt, rpt)],
          )

      if ph + 1 < nph:
        plsc.subcore_barrier()

  return sc_layer


def kernel(x, edge_index, W1, att_src1, att_dst1, b1, W2, att_src2,
           att_dst2, b2):
  n, d = x.shape
  e = edge_index.shape[1]
  h, f = att_src1.shape
  c = W2.shape[1]
  fp2 = 48

  src = edge_index[0].reshape(e // _KC, _KC)
  dst = edge_index[1].reshape(e // _KC, _KC)

  eye_h = jnp.eye(h, dtype=jnp.float32)
  a1s = (eye_h[:, None, :] * att_src1[:, :, None]).reshape(h * f, h)
  a1d = (eye_h[:, None, :] * att_dst1[:, :, None]).reshape(h * f, h)
  acat1 = jnp.concatenate([a1s, a1d], axis=1)

  acat2 = jnp.zeros((fp2, 16), jnp.float32)
  acat2 = acat2.at[:c, 0].set(att_src2[0])
  acat2 = acat2.at[:c, 1].set(att_dst2[0])
  W2p = jnp.zeros((h * f, fp2), jnp.float32).at[:, :c].set(W2)
  sel = jnp.eye(fp2, c, dtype=jnp.float32)

  zbig1 = jnp.zeros((n, 64), jnp.float32)
  zden1 = jnp.zeros((n, h), jnp.float32)
  zbig2 = jnp.zeros((n, fp2), jnp.float32)
  zden2 = jnp.zeros((n,), jnp.float32)

  hlo, hhi, a1 = _tc1(x, W1, acat1)
  sc1 = _make_sc_gat(n, e, h, 64, 2)
  *parts1, _, _, _, _ = sc1(hlo, hhi, a1, src, dst, zbig1, zden1)

  h2, a2 = _tc2(parts1, b1.reshape(1, -1), W2p, acat2)
  sc2 = _make_sc_gat(n, e, 1, fp2, 1)
  p20, p21, *_ = sc2(h2, a2, src, dst, zbig2, zden2)

  return _tc3(p20, p21, b2.reshape(1, -1), sel)


# speedup vs baseline: 62.2003x; 1.0290x over previous
import functools

import jax
import jax.numpy as jnp
from jax import lax
from jax.experimental import pallas as pl
from jax.experimental.pallas import tpu as pltpu
from jax.experimental.pallas import tpu_sc as plsc

_NEG = 0.2
_EPS = 1e-16
_L = 16
_KC = 80
_KR = 5
_K = _KR * _KC


def _tc1_body(x_ref, w1_ref, acat_ref, h0_ref, h1_ref, a_ref):
  h = jnp.dot(x_ref[...], w1_ref[...], preferred_element_type=jnp.float32)
  h0_ref[...] = h[:, 0:64]
  h1_ref[...] = h[:, 64:128]
  a_ref[...] = jnp.dot(h, acat_ref[...], preferred_element_type=jnp.float32)


def _tc2_body(o00_ref, o01_ref, o10_ref, o11_ref, b1_ref, w2_ref, acat_ref,
              h2_ref, a2_ref):
  o = jnp.concatenate(
      [o00_ref[...] + o10_ref[...], o01_ref[...] + o11_ref[...]], axis=1
  ) + b1_ref[...]
  he = jnp.where(o > 0.0, o, jnp.exp(o) - 1.0)
  h2 = jnp.dot(he, w2_ref[...], preferred_element_type=jnp.float32)
  h2_ref[...] = h2
  a2_ref[...] = jnp.dot(h2, acat_ref[...], preferred_element_type=jnp.float32)


def _tc3_body(oa_ref, ob_ref, b2_ref, sel_ref, out_ref):
  o = oa_ref[...] + ob_ref[...]
  out_ref[...] = (
      jnp.dot(o, sel_ref[...], preferred_element_type=jnp.float32)
      + b2_ref[...]
  )


def _tc1(x, W1, acat):
  n = x.shape[0]
  d = x.shape[1]
  b = 1000
  return pl.pallas_call(
      _tc1_body,
      grid=(n // b,),
      in_specs=[
          pl.BlockSpec((b, d), lambda i: (i, 0)),
          pl.BlockSpec((d, 128), lambda i: (0, 0)),
          pl.BlockSpec((128, 16), lambda i: (0, 0)),
      ],
      out_specs=[pl.BlockSpec((b, 64), lambda i: (i, 0))] * 2
      + [pl.BlockSpec((b, 16), lambda i: (i, 0))],
      out_shape=[jax.ShapeDtypeStruct((n, 64), jnp.float32)] * 2
      + [jax.ShapeDtypeStruct((n, 16), jnp.float32)],
  )(x, W1, acat)


def _tc2(parts, b1, W2p, acat2):
  n = parts[0].shape[0]
  fp = W2p.shape[1]
  b = 1000
  quarter = pl.BlockSpec((b, 64), lambda i: (i, 0))
  return pl.pallas_call(
      _tc2_body,
      grid=(n // b,),
      in_specs=[quarter] * 4
      + [
          pl.BlockSpec((1, 128), lambda i: (0, 0)),
          pl.BlockSpec((128, fp), lambda i: (0, 0)),
          pl.BlockSpec((fp, 16), lambda i: (0, 0)),
      ],
      out_specs=[
          pl.BlockSpec((b, fp), lambda i: (i, 0)),
          pl.BlockSpec((b, 16), lambda i: (i, 0)),
      ],
      out_shape=[
          jax.ShapeDtypeStruct((n, fp), jnp.float32),
          jax.ShapeDtypeStruct((n, 16), jnp.float32),
      ],
  )(*parts, b1, W2p, acat2)


def _tc3(oa, ob, b2, sel):
  n = oa.shape[0]
  fp = oa.shape[1]
  c = sel.shape[1]
  b = 1000
  return pl.pallas_call(
      _tc3_body,
      grid=(n // b,),
      in_specs=[
          pl.BlockSpec((b, fp), lambda i: (i, 0)),
          pl.BlockSpec((b, fp), lambda i: (i, 0)),
          pl.BlockSpec((1, c), lambda i: (0, 0)),
          pl.BlockSpec((fp, c), lambda i: (0, 0)),
      ],
      out_specs=pl.BlockSpec((b, c), lambda i: (i, 0)),
      out_shape=jax.ShapeDtypeStruct((n, c), jnp.float32),
  )(oa, ob, b2, sel)


def _make_sc_gat(n, e, hh, fph, nph):
  info = plsc.get_sparse_core_info()
  nc, ns = info.num_cores, info.num_subcores
  nw = nc * ns
  hph = hh // nph
  ea = e // ns
  eb = e // nw
  ca = ea // _K
  cb = eb // _K
  nstr = 10
  rpt = n // nstr
  assert ea % _K == 0 and eb % _K == 0 and n % nstr == 0 and rpt % 8 == 0

  mesh = plsc.VectorSubcoreMesh(core_axis_name="c", subcore_axis_name="s")
  den_shape = (n, hh) if hh > 1 else (n,)
  w_shape = (_K, hh) if hh > 1 else (_K,)

  @functools.partial(
      pl.kernel,
      out_type=[
          jax.ShapeDtypeStruct((n, fph), jnp.float32)
          for _ in range(nc * nph)
      ]
      + [jax.ShapeDtypeStruct(den_shape, jnp.float32) for _ in range(nc)]
      + [
          jax.ShapeDtypeStruct((e, hh) if hh > 1 else (e,), jnp.float32)
          for _ in range(nc)
      ],
      mesh=mesh,
      compiler_params=pltpu.CompilerParams(
          needs_layout_passes=False, use_tc_tiling_on_sc=False
      ),
      scratch_types=[
          pltpu.VMEM((_KR, _KC), jnp.int32),
          pltpu.VMEM((_KR, _KC), jnp.int32),
          pltpu.VMEM((_K, 16), jnp.float32),
          pltpu.VMEM((_K, 16), jnp.float32),
          pltpu.VMEM(w_shape, jnp.float32),
          pltpu.VMEM(w_shape, jnp.float32),
          pltpu.VMEM((_K, fph), jnp.float32),
          pltpu.VMEM(w_shape, jnp.float32),
          pltpu.VMEM_SHARED(den_shape, jnp.float32),
          pltpu.VMEM_SHARED((n, fph), jnp.float32),
          pltpu.SemaphoreType.DMA,
          pltpu.SemaphoreType.DMA,
      ],
  )
  def sc_layer(*refs):
    h_hbms = refs[:nph]
    a_hbm, src_hbm, dst_hbm, zbig_hbm, zden_hbm = refs[nph:nph + 5]
    out_hbms = refs[nph + 5:nph + 5 + nc * nph]
    den_hbms = refs[nph + 5 + nc * nph:nph + 5 + nc * nph + nc]
    w_hbms = refs[nph + 5 + nc * (nph + 1):nph + 5 + nc * (nph + 2)]
    (srcidx_v, dstidx_v, asrc_v, adst_v, w_v, alpha_v, hrows_v,
     den_rows_v, den_sp, out_sp, sem, semh) = refs[nph + 5 + nc * (nph + 2):]

    def drain(cps):
      for cp in cps:
        cp.wait()

    def load_idx(row0):
      c1 = pltpu.async_copy(src_hbm.at[pl.ds(row0, _KR)], srcidx_v, sem)
      c2 = pltpu.async_copy(dst_hbm.at[pl.ds(row0, _KR)], dstidx_v, sem)
      drain([c1, c2])

    def gather_coeffs():
      cps = []
      for j in range(_KR):
        sl = pl.ds(j * _KC, _KC)
        cps.append(
            pltpu.async_copy(a_hbm.at[srcidx_v.at[j]], asrc_v.at[sl], sem)
        )
        cps.append(
            pltpu.async_copy(a_hbm.at[dstidx_v.at[j]], adst_v.at[sl], sem)
        )
      return cps

    def scatter_rows(src_ref, dst_ref):
      cps = []
      for j in range(_KR):
        cps.append(
            pltpu.async_copy(
                src_ref.at[pl.ds(j * _KC, _KC)],
                dst_ref.at[dstidx_v.at[j]],
                sem,
                add=True,
            )
        )
      return cps

    cid = lax.axis_index("c")
    sid = lax.axis_index("s")
    wid = sid * nc + cid
    iota = lax.iota(jnp.int32, _L)
    row2 = iota >> 3
    col8 = iota & 7
    z16 = iota * 0
    o16 = z16 + 1

    def zero_stripes(sp_ref, z_ref):
      @pl.when(sid < nstr)
      def _():
        pltpu.sync_copy(
            z_ref.at[pl.ds(sid * rpt, rpt)], sp_ref.at[pl.ds(sid * rpt, rpt)]
        )

    def leaky_exp(a_s, a_d):
      ee = a_s + a_d
      return jnp.exp(jnp.where(ee > 0.0, ee, _NEG * ee))

    zero_stripes(den_sp, zden_hbm)
    plsc.subcore_barrier()

    def w_chunk():
      if hh > 1:
        @plsc.parallel_loop(0, _K // 2, unroll=4)
        def _(p):
          rows = p * 2 + row2
          a_s = plsc.load_gather(asrc_v, [rows, col8])
          a_d = plsc.load_gather(adst_v, [rows, col8 + 8])
          plsc.store_scatter(w_v, [rows, col8], leaky_exp(a_s, a_d))
      else:
        @plsc.parallel_loop(0, _K // _L, unroll=4)
        def _(g):
          rows = g * _L + iota
          a_s = plsc.load_gather(asrc_v, [rows, z16])
          a_d = plsc.load_gather(adst_v, [rows, o16])
          plsc.store_scatter(w_v, [rows], leaky_exp(a_s, a_d))

    def pass_a(i, carry):
      load_idx(sid * (ea // _KC) + i * _KR)
      drain(gather_coeffs())
      w_chunk()
      for cc in range(nc):
        @pl.when(cid == cc)
        def _(cc=cc):
          ebase = sid * ea + i * _K
          pltpu.async_copy(
              w_v, w_hbms[cc].at[pl.ds(ebase, _K)], sem
          ).wait()
      drain(scatter_rows(w_v, den_sp))
      return carry

    lax.fori_loop(0, ca, pass_a, 0)
    plsc.subcore_barrier()

    for cc in range(nc):
      @pl.when((sid < nstr) & (cid == cc))
      def _(cc=cc):
        pltpu.sync_copy(
            den_sp.at[pl.ds(sid * rpt, rpt)],
            den_hbms[cc].at[pl.ds(sid * rpt, rpt)],
        )
    plsc.subcore_barrier()

    for ph in range(nph):
      zero_stripes(out_sp, zbig_hbm)
      plsc.subcore_barrier()

      def pass_b(i, carry):
        ebase = wid * eb + i * _K
        load_idx(wid * (eb // _KC) + i * _KR)
        cps = []
        for j in range(_KR):
          cps.append(
              pltpu.async_copy(
                  h_hbms[ph].at[srcidx_v.at[j]],
                  hrows_v.at[pl.ds(j * _KC, _KC)],
                  semh,
              )
          )
        for cc in range(nc):
          @pl.when(cid == cc)
          def _(cc=cc):
            dcps = [
                pltpu.async_copy(
                    w_hbms[cc].at[pl.ds(ebase, _K)], w_v, sem
                )
            ]
            for j in range(_KR):
              dcps.append(
                  pltpu.async_copy(
                      den_hbms[cc].at[dstidx_v.at[j]],
                      den_rows_v.at[pl.ds(j * _KC, _KC)],
                      sem,
                  )
              )
            drain(dcps)
        drain(cps)
        if hh > 1:
          @plsc.parallel_loop(0, _K // 2, unroll=4)
          def _(p):
            rows = p * 2 + row2
            w = plsc.load_gather(w_v, [rows, col8])
            den = plsc.load_gather(den_rows_v, [rows, col8])
            plsc.store_scatter(alpha_v, [rows, col8], w / (den + _EPS))
        else:
          @plsc.parallel_loop(0, _K // _L, unroll=4)
          def _(g):
            rows = g * _L + iota
            w = plsc.load_gather(w_v, [rows])
            den = plsc.load_gather(den_rows_v, [rows])
            plsc.store_scatter(alpha_v, [rows], w / (den + _EPS))

        @plsc.parallel_loop(0, _K, unroll=8)
        def _(ei):
          ei_v = z16 + ei
          if hh > 1:
            for head in range(hph):
              a = plsc.load_gather(alpha_v, [ei_v, z16 + (ph * hph + head)])
              sl = pl.ds(head * _L, _L)
              hrows_v[ei, sl] = hrows_v[ei, sl] * a
          else:
            a = plsc.load_gather(alpha_v, [ei_v])
            for j in range(fph // _L):
              sl = pl.ds(j * _L, _L)
              hrows_v[ei, sl] = hrows_v[ei, sl] * a
        drain(scatter_rows(hrows_v, out_sp))
        return carry

      lax.fori_loop(0, cb, pass_b, 0)
      plsc.subcore_barrier()

      for cc in range(nc):
        @pl.when((sid < nstr) & (cid == cc))
        def _():
          pltpu.sync_copy(
              out_sp.at[pl.ds(sid * rpt, rpt)],
              out_hbms[cc * nph + ph].at[pl.ds(sid * rpt, rpt)],
          )

      if ph + 1 < nph:
        plsc.subcore_barrier()

  return sc_layer


def kernel(x, edge_index, W1, att_src1, att_dst1, b1, W2, att_src2,
           att_dst2, b2):
  n, d = x.shape
  e = edge_index.shape[1]
  h, f = att_src1.shape
  c = W2.shape[1]
  fp2 = 48

  src = edge_index[0].reshape(e // _KC, _KC)
  dst = edge_index[1].reshape(e // _KC, _KC)

  eye_h = jnp.eye(h, dtype=jnp.float32)
  a1s = (eye_h[:, None, :] * att_src1[:, :, None]).reshape(h * f, h)
  a1d = (eye_h[:, None, :] * att_dst1[:, :, None]).reshape(h * f, h)
  acat1 = jnp.concatenate([a1s, a1d], axis=1)

  acat2 = jnp.zeros((fp2, 16), jnp.float32)
  acat2 = acat2.at[:c, 0].set(att_src2[0])
  acat2 = acat2.at[:c, 1].set(att_dst2[0])
  W2p = jnp.zeros((h * f, fp2), jnp.float32).at[:, :c].set(W2)
  sel = jnp.eye(fp2, c, dtype=jnp.float32)

  zbig1 = jnp.zeros((n, 64), jnp.float32)
  zden1 = jnp.zeros((n, h), jnp.float32)
  zbig2 = jnp.zeros((n, fp2), jnp.float32)
  zden2 = jnp.zeros((n,), jnp.float32)

  hlo, hhi, a1 = _tc1(x, W1, acat1)
  sc1 = _make_sc_gat(n, e, h, 64, 2)
  *parts1, _, _, _, _ = sc1(hlo, hhi, a1, src, dst, zbig1, zden1)

  h2, a2 = _tc2(parts1, b1.reshape(1, -1), W2p, acat2)
  sc2 = _make_sc_gat(n, e, 1, fp2, 1)
  p20, p21, *_ = sc2(h2, a2, src, dst, zbig2, zden2)

  return _tc3(p20, p21, b2.reshape(1, -1), sel)


# speedup vs baseline: 62.2491x; 1.0008x over previous
import functools

import jax
import jax.numpy as jnp
from jax import lax
from jax.experimental import pallas as pl
from jax.experimental.pallas import tpu as pltpu
from jax.experimental.pallas import tpu_sc as plsc

_NEG = 0.2
_EPS = 1e-16
_L = 16
_KC = 80
_KR = 5
_K = _KR * _KC


def _tc1_body(x_ref, w1_ref, acat_ref, h0_ref, h1_ref, a_ref):
  h = jnp.dot(x_ref[...], w1_ref[...], preferred_element_type=jnp.float32)
  h0_ref[...] = h[:, 0:64]
  h1_ref[...] = h[:, 64:128]
  a_ref[...] = jnp.dot(h, acat_ref[...], preferred_element_type=jnp.float32)


def _tc2_body(o00_ref, o01_ref, o10_ref, o11_ref, b1_ref, w2_ref, acat_ref,
              h2_ref, a2_ref):
  o = jnp.concatenate(
      [o00_ref[...] + o10_ref[...], o01_ref[...] + o11_ref[...]], axis=1
  ) + b1_ref[...]
  he = jnp.where(o > 0.0, o, jnp.exp(o) - 1.0)
  h2 = jnp.dot(he, w2_ref[...], preferred_element_type=jnp.float32)
  h2_ref[...] = h2
  a2_ref[...] = jnp.dot(h2, acat_ref[...], preferred_element_type=jnp.float32)


def _tc3_body(oa_ref, ob_ref, b2_ref, sel_ref, out_ref):
  o = oa_ref[...] + ob_ref[...]
  out_ref[...] = (
      jnp.dot(o, sel_ref[...], preferred_element_type=jnp.float32)
      + b2_ref[...]
  )


def _tc1(x, W1, acat):
  n = x.shape[0]
  d = x.shape[1]
  b = 1000
  return pl.pallas_call(
      _tc1_body,
      grid=(n // b,),
      in_specs=[
          pl.BlockSpec((b, d), lambda i: (i, 0)),
          pl.BlockSpec((d, 128), lambda i: (0, 0)),
          pl.BlockSpec((128, 16), lambda i: (0, 0)),
      ],
      out_specs=[pl.BlockSpec((b, 64), lambda i: (i, 0))] * 2
      + [pl.BlockSpec((b, 16), lambda i: (i, 0))],
      out_shape=[jax.ShapeDtypeStruct((n, 64), jnp.float32)] * 2
      + [jax.ShapeDtypeStruct((n, 16), jnp.float32)],
  )(x, W1, acat)


def _tc2(parts, b1, W2p, acat2):
  n = parts[0].shape[0]
  fp = W2p.shape[1]
  b = 1000
  quarter = pl.BlockSpec((b, 64), lambda i: (i, 0))
  return pl.pallas_call(
      _tc2_body,
      grid=(n // b,),
      in_specs=[quarter] * 4
      + [
          pl.BlockSpec((1, 128), lambda i: (0, 0)),
          pl.BlockSpec((128, fp), lambda i: (0, 0)),
          pl.BlockSpec((fp, 16), lambda i: (0, 0)),
      ],
      out_specs=[
          pl.BlockSpec((b, fp), lambda i: (i, 0)),
          pl.BlockSpec((b, 16), lambda i: (i, 0)),
      ],
      out_shape=[
          jax.ShapeDtypeStruct((n, fp), jnp.float32),
          jax.ShapeDtypeStruct((n, 16), jnp.float32),
      ],
  )(*parts, b1, W2p, acat2)


def _tc3(oa, ob, b2, sel):
  n = oa.shape[0]
  fp = oa.shape[1]
  c = sel.shape[1]
  b = 1000
  return pl.pallas_call(
      _tc3_body,
      grid=(n // b,),
      in_specs=[
          pl.BlockSpec((b, fp), lambda i: (i, 0)),
          pl.BlockSpec((b, fp), lambda i: (i, 0)),
          pl.BlockSpec((1, c), lambda i: (0, 0)),
          pl.BlockSpec((fp, c), lambda i: (0, 0)),
      ],
      out_specs=pl.BlockSpec((b, c), lambda i: (i, 0)),
      out_shape=jax.ShapeDtypeStruct((n, c), jnp.float32),
  )(oa, ob, b2, sel)


def _make_sc_gat(n, e, hh, fph, nph):
  info = plsc.get_sparse_core_info()
  nc, ns = info.num_cores, info.num_subcores
  nw = nc * ns
  hph = hh // nph
  ea = e // ns
  eb = e // nw
  ca = ea // _K
  cb = eb // _K
  nstr = 10
  rpt = n // nstr
  assert ea % _K == 0 and eb % _K == 0 and n % nstr == 0 and rpt % 8 == 0

  mesh = plsc.VectorSubcoreMesh(core_axis_name="c", subcore_axis_name="s")
  den_shape = (n, hh) if hh > 1 else (n,)
  w_shape = (_K, hh) if hh > 1 else (_K,)

  @functools.partial(
      pl.kernel,
      out_type=[
          jax.ShapeDtypeStruct((n, fph), jnp.float32)
          for _ in range(nc * nph)
      ]
      + [jax.ShapeDtypeStruct(den_shape, jnp.float32) for _ in range(nc)]
      + [
          jax.ShapeDtypeStruct((e, hh) if hh > 1 else (e,), jnp.float32)
          for _ in range(nc)
      ],
      mesh=mesh,
      compiler_params=pltpu.CompilerParams(
          needs_layout_passes=False, use_tc_tiling_on_sc=False
      ),
      scratch_types=[
          pltpu.VMEM((_KR, _KC), jnp.int32),
          pltpu.VMEM((_KR, _KC), jnp.int32),
          pltpu.VMEM((_K, 16), jnp.float32),
          pltpu.VMEM((_K, 16), jnp.float32),
          pltpu.VMEM(w_shape, jnp.float32),
          pltpu.VMEM(w_shape, jnp.float32),
          pltpu.VMEM((_K, fph), jnp.float32),
          pltpu.VMEM(w_shape, jnp.float32),
          pltpu.VMEM_SHARED(den_shape, jnp.float32),
          pltpu.VMEM_SHARED((n, fph), jnp.float32),
          pltpu.SemaphoreType.DMA,
          pltpu.SemaphoreType.DMA,
      ],
  )
  def sc_layer(*refs):
    h_hbms = refs[:nph]
    a_hbm, src_hbm, dst_hbm, zbig_hbm, zden_hbm = refs[nph:nph + 5]
    out_hbms = refs[nph + 5:nph + 5 + nc * nph]
    den_hbms = refs[nph + 5 + nc * nph:nph + 5 + nc * nph + nc]
    w_hbms = refs[nph + 5 + nc * (nph + 1):nph + 5 + nc * (nph + 2)]
    (srcidx_v, dstidx_v, asrc_v, adst_v, w_v, alpha_v, hrows_v,
     den_rows_v, den_sp, out_sp, sem, semh) = refs[nph + 5 + nc * (nph + 2):]

    def drain(cps):
      for cp in cps:
        cp.wait()

    def load_idx(row0):
      c1 = pltpu.async_copy(src_hbm.at[pl.ds(row0, _KR)], srcidx_v, sem)
      c2 = pltpu.async_copy(dst_hbm.at[pl.ds(row0, _KR)], dstidx_v, sem)
      drain([c1, c2])

    def gather_coeffs():
      cps = []
      for j in range(_KR):
        sl = pl.ds(j * _KC, _KC)
        cps.append(
            pltpu.async_copy(a_hbm.at[srcidx_v.at[j]], asrc_v.at[sl], sem)
        )
        cps.append(
            pltpu.async_copy(a_hbm.at[dstidx_v.at[j]], adst_v.at[sl], sem)
        )
      return cps

    def scatter_rows(src_ref, dst_ref):
      cps = []
      for j in range(_KR):
        cps.append(
            pltpu.async_copy(
                src_ref.at[pl.ds(j * _KC, _KC)],
                dst_ref.at[dstidx_v.at[j]],
                sem,
                add=True,
            )
        )
      return cps

    cid = lax.axis_index("c")
    sid = lax.axis_index("s")
    wid = sid * nc + cid
    iota = lax.iota(jnp.int32, _L)
    row2 = iota >> 3
    col8 = iota & 7
    z16 = iota * 0
    o16 = z16 + 1

    def zero_stripes(sp_ref, z_ref):
      @pl.when(sid < nstr)
      def _():
        pltpu.sync_copy(
            z_ref.at[pl.ds(sid * rpt, rpt)], sp_ref.at[pl.ds(sid * rpt, rpt)]
        )

    def leaky_exp(a_s, a_d):
      ee = a_s + a_d
      return jnp.exp(jnp.where(ee > 0.0, ee, _NEG * ee))

    zero_stripes(den_sp, zden_hbm)
    plsc.subcore_barrier()

    def w_chunk():
      if hh > 1:
        @plsc.parallel_loop(0, _K // 2, unroll=8)
        def _(p):
          rows = p * 2 + row2
          a_s = plsc.load_gather(asrc_v, [rows, col8])
          a_d = plsc.load_gather(adst_v, [rows, col8 + 8])
          plsc.store_scatter(w_v, [rows, col8], leaky_exp(a_s, a_d))
      else:
        @plsc.parallel_loop(0, _K // _L, unroll=8)
        def _(g):
          rows = g * _L + iota
          a_s = plsc.load_gather(asrc_v, [rows, z16])
          a_d = plsc.load_gather(adst_v, [rows, o16])
          plsc.store_scatter(w_v, [rows], leaky_exp(a_s, a_d))

    def pass_a(i, carry):
      load_idx(sid * (ea // _KC) + i * _KR)
      drain(gather_coeffs())
      w_chunk()
      for cc in range(nc):
        @pl.when(cid == cc)
        def _(cc=cc):
          ebase = sid * ea + i * _K
          pltpu.async_copy(
              w_v, w_hbms[cc].at[pl.ds(ebase, _K)], sem
          ).wait()
      drain(scatter_rows(w_v, den_sp))
      return carry

    lax.fori_loop(0, ca, pass_a, 0)
    plsc.subcore_barrier()

    for cc in range(nc):
      @pl.when((sid < nstr) & (cid == cc))
      def _(cc=cc):
        pltpu.sync_copy(
            den_sp.at[pl.ds(sid * rpt, rpt)],
            den_hbms[cc].at[pl.ds(sid * rpt, rpt)],
        )
    plsc.subcore_barrier()

    for ph in range(nph):
      zero_stripes(out_sp, zbig_hbm)
      plsc.subcore_barrier()

      def pass_b(i, carry):
        ebase = wid * eb + i * _K
        load_idx(wid * (eb // _KC) + i * _KR)
        cps = []
        for j in range(_KR):
          cps.append(
              pltpu.async_copy(
                  h_hbms[ph].at[srcidx_v.at[j]],
                  hrows_v.at[pl.ds(j * _KC, _KC)],
                  semh,
              )
          )
        for cc in range(nc):
          @pl.when(cid == cc)
          def _(cc=cc):
            dcps = [
                pltpu.async_copy(
                    w_hbms[cc].at[pl.ds(ebase, _K)], w_v, sem
                )
            ]
            for j in range(_KR):
              dcps.append(
                  pltpu.async_copy(
                      den_hbms[cc].at[dstidx_v.at[j]],
                      den_rows_v.at[pl.ds(j * _KC, _KC)],
                      sem,
                  )
              )
            drain(dcps)
        drain(cps)
        if hh > 1:
          @plsc.parallel_loop(0, _K // 2, unroll=8)
          def _(p):
            rows = p * 2 + row2
            w = plsc.load_gather(w_v, [rows, col8])
            den = plsc.load_gather(den_rows_v, [rows, col8])
            plsc.store_scatter(alpha_v, [rows, col8], w / (den + _EPS))
        else:
          @plsc.parallel_loop(0, _K // _L, unroll=8)
          def _(g):
            rows = g * _L + iota
            w = plsc.load_gather(w_v, [rows])
            den = plsc.load_gather(den_rows_v, [rows])
            plsc.store_scatter(alpha_v, [rows], w / (den + _EPS))

        @plsc.parallel_loop(0, _K, unroll=8)
        def _(ei):
          ei_v = z16 + ei
          if hh > 1:
            for head in range(hph):
              a = plsc.load_gather(alpha_v, [ei_v, z16 + (ph * hph + head)])
              sl = pl.ds(head * _L, _L)
              hrows_v[ei, sl] = hrows_v[ei, sl] * a
          else:
            a = plsc.load_gather(alpha_v, [ei_v])
            for j in range(fph // _L):
              sl = pl.ds(j * _L, _L)
              hrows_v[ei, sl] = hrows_v[ei, sl] * a
        drain(scatter_rows(hrows_v, out_sp))
        return carry

      lax.fori_loop(0, cb, pass_b, 0)
      plsc.subcore_barrier()

      for cc in range(nc):
        @pl.when((sid < nstr) & (cid == cc))
        def _():
          pltpu.sync_copy(
              out_sp.at[pl.ds(sid * rpt, rpt)],
              out_hbms[cc * nph + ph].at[pl.ds(sid * rpt, rpt)],
          )

      if ph + 1 < nph:
        plsc.subcore_barrier()

  return sc_layer


def kernel(x, edge_index, W1, att_src1, att_dst1, b1, W2, att_src2,
           att_dst2, b2):
  n, d = x.shape
  e = edge_index.shape[1]
  h, f = att_src1.shape
  c = W2.shape[1]
  fp2 = 48

  src = edge_index[0].reshape(e // _KC, _KC)
  dst = edge_index[1].reshape(e // _KC, _KC)

  eye_h = jnp.eye(h, dtype=jnp.float32)
  a1s = (eye_h[:, None, :] * att_src1[:, :, None]).reshape(h * f, h)
  a1d = (eye_h[:, None, :] * att_dst1[:, :, None]).reshape(h * f, h)
  acat1 = jnp.concatenate([a1s, a1d], axis=1)

  acat2 = jnp.zeros((fp2, 16), jnp.float32)
  acat2 = acat2.at[:c, 0].set(att_src2[0])
  acat2 = acat2.at[:c, 1].set(att_dst2[0])
  W2p = jnp.zeros((h * f, fp2), jnp.float32).at[:, :c].set(W2)
  sel = jnp.eye(fp2, c, dtype=jnp.float32)

  zbig1 = jnp.zeros((n, 64), jnp.float32)
  zden1 = jnp.zeros((n, h), jnp.float32)
  zbig2 = jnp.zeros((n, fp2), jnp.float32)
  zden2 = jnp.zeros((n,), jnp.float32)

  hlo, hhi, a1 = _tc1(x, W1, acat1)
  sc1 = _make_sc_gat(n, e, h, 64, 2)
  *parts1, _, _, _, _ = sc1(hlo, hhi, a1, src, dst, zbig1, zden1)

  h2, a2 = _tc2(parts1, b1.reshape(1, -1), W2p, acat2)
  sc2 = _make_sc_gat(n, e, 1, fp2, 1)
  p20, p21, *_ = sc2(h2, a2, src, dst, zbig2, zden2)

  return _tc3(p20, p21, b2.reshape(1, -1), sel)
